# baseline (speedup 1.0000x reference)
import jax
import jax.numpy as jnp
from jax import lax
from jax.experimental import pallas as pl
from jax.experimental.pallas import tpu as pltpu
from jax.experimental.pallas import tpu_sc as plsc

PROMPT_NUM = 50
HIDDEN_SIZE = 1024
BATCH = 1024

_NC = 2
_NS = 16
_NW = _NC * _NS
_ROWS_PER_W = BATCH // _NW
_MASK_PER_W = _ROWS_PER_W * PROMPT_NUM


def _sc_body(table_hbm, emb_hbm, mask_hbm, table_v, ones_v, sem):
    wid = lax.axis_index("s") * _NC + lax.axis_index("c")
    base = wid * _ROWS_PER_W

    pltpu.sync_copy(table_hbm, table_v)

    def _fill(i, carry):
        ones_v[pl.ds(i * 16, 16)] = jnp.ones((16,), jnp.float32)
        return carry

    lax.fori_loop(0, _MASK_PER_W // 16, _fill, 0)

    handles = [
        pltpu.async_copy(table_v, emb_hbm.at[base + r], sem)
        for r in range(_ROWS_PER_W)
    ]
    pltpu.sync_copy(ones_v, mask_hbm.at[pl.ds(wid * _MASK_PER_W, _MASK_PER_W)])
    for h in handles:
        h.wait()


def _sc_broadcast(prompt_table):
    mesh = plsc.VectorSubcoreMesh(core_axis_name="c", subcore_axis_name="s")
    emb, mask_flat = pl.kernel(
        _sc_body,
        out_type=(
            jax.ShapeDtypeStruct((BATCH, PROMPT_NUM, HIDDEN_SIZE), jnp.float32),
            jax.ShapeDtypeStruct((BATCH * PROMPT_NUM,), jnp.float32),
        ),
        mesh=mesh,
        scratch_types=[
            pltpu.VMEM((PROMPT_NUM, HIDDEN_SIZE), jnp.float32),
            pltpu.VMEM((_MASK_PER_W,), jnp.float32),
            pltpu.SemaphoreType.DMA,
        ],
    )(prompt_table)
    return emb, mask_flat


def kernel(batch_size, prompt_table):
    emb, mask_flat = _sc_broadcast(prompt_table)
    return emb, mask_flat.reshape(BATCH, PROMPT_NUM)


# speedup vs baseline: 1.1344x; 1.1344x over previous
import jax
import jax.numpy as jnp
from jax import lax
from jax.experimental import pallas as pl
from jax.experimental.pallas import tpu as pltpu
from jax.experimental.pallas import tpu_sc as plsc

PROMPT_NUM = 50
HIDDEN_SIZE = 1024
BATCH = 1024

_K = 16
_NDMA = BATCH // _K


def _tc_body(table_v, emb_hbm, mask_hbm, staged, ones_v, sem):
    staged[...] = jnp.broadcast_to(
        table_v[...][None], (_K, PROMPT_NUM, HIDDEN_SIZE)
    )
    ones_v[...] = jnp.ones((BATCH, PROMPT_NUM), jnp.float32)
    handles = [
        pltpu.make_async_copy(staged, emb_hbm.at[pl.ds(j * _K, _K)], sem)
        for j in range(_NDMA)
    ]
    mask_h = pltpu.make_async_copy(ones_v, mask_hbm, sem)
    for h in handles:
        h.start()
    mask_h.start()
    for h in handles:
        h.wait()
    mask_h.wait()


def _tc_broadcast(prompt_table):
    return pl.pallas_call(
        _tc_body,
        out_shape=(
            jax.ShapeDtypeStruct((BATCH, PROMPT_NUM, HIDDEN_SIZE), jnp.float32),
            jax.ShapeDtypeStruct((BATCH, PROMPT_NUM), jnp.float32),
        ),
        in_specs=[pl.BlockSpec(memory_space=pltpu.VMEM)],
        out_specs=(
            pl.BlockSpec(memory_space=pl.ANY),
            pl.BlockSpec(memory_space=pl.ANY),
        ),
        scratch_shapes=[
            pltpu.VMEM((_K, PROMPT_NUM, HIDDEN_SIZE), jnp.float32),
            pltpu.VMEM((BATCH, PROMPT_NUM), jnp.float32),
            pltpu.SemaphoreType.DMA,
        ],
    )(prompt_table)


def kernel(batch_size, prompt_table):
    emb, mask = _tc_broadcast(prompt_table)
    return emb, mask
